# Initial kernel scaffold; baseline (speedup 1.0000x reference)
import jax
import jax.numpy as jnp
from jax.experimental import pallas as pl


def kernel(input_seqs, input_lengths, table, w_ih_f, w_hh_f, b_ih_f, b_hh_f, w_ih_b, w_hh_b, b_ih_b, b_hh_b):
    raise NotImplementedError("write your pallas kernel here")


# speedup vs baseline: 1.3430x; 1.3430x over previous
import functools

import jax
import jax.numpy as jnp
from jax import lax
from jax.experimental import pallas as pl
from jax.experimental.pallas import tpu as pltpu
from jax.experimental.pallas import tpu_sc as plsc


def _sc_gather(table, idx):
    V, H = table.shape
    N = idx.shape[0]
    info = plsc.get_sparse_core_info()
    NC, NS = info.num_cores, info.num_subcores
    NW = NC * NS
    assert N % NW == 0
    per_w = N // NW
    CH = 128
    sizes = [CH] * (per_w // CH)
    if per_w % CH:
        sizes.append(per_w % CH)
    offs = [0]
    for s in sizes[:-1]:
        offs.append(offs[-1] + s)
    n_ch = len(sizes)

    mesh = plsc.VectorSubcoreMesh(core_axis_name="c", subcore_axis_name="s")

    @functools.partial(
        pl.kernel,
        mesh=mesh,
        out_type=jax.ShapeDtypeStruct((N, H), jnp.float32),
        scratch_types=[
            pltpu.VMEM((per_w,), jnp.int32),
            pltpu.VMEM((CH, H), jnp.float32),
            pltpu.VMEM((CH, H), jnp.float32),
            pltpu.SemaphoreType.DMA,
            pltpu.SemaphoreType.DMA,
        ],
    )
    def gather_kernel(table_hbm, idx_hbm, out_hbm, idx_v, rows0, rows1, sem0, sem1):
        wid = lax.axis_index("s") * NC + lax.axis_index("c")
        base = wid * per_w
        pltpu.sync_copy(idx_hbm.at[pl.ds(base, per_w)], idx_v)
        bufs = (rows0, rows1)
        sems = (sem0, sem1)

        def start(j):
            buf = bufs[j % 2].at[pl.ds(0, sizes[j])]
            return pltpu.async_copy(
                table_hbm.at[idx_v.at[pl.ds(offs[j], sizes[j])]], buf, sems[j % 2]
            )

        def drain(j, cp):
            cp.wait()
            pltpu.sync_copy(
                bufs[j % 2].at[pl.ds(0, sizes[j])],
                out_hbm.at[pl.ds(base + offs[j], sizes[j])],
            )

        cp = start(0)
        for j in range(1, n_ch):
            cp_next = start(j)
            drain(j - 1, cp)
            cp = cp_next
        drain(n_ch - 1, cp)

    return gather_kernel(table, idx)


def _gru_bidir_tc(emb, w_ih_t, w_hh_t, b_ih, b_hh):
    T, B, HP = emb.shape
    H = w_hh_t.shape[1]
    H3 = 3 * H

    def body(emb_ref, wih_ref, whh_ref, bih_ref, bhh_ref, out_ref, hid_ref, h_scr):
        i = pl.program_id(0)

        @pl.when((i == 0) | (i == T))
        def _():
            h_scr[...] = jnp.zeros_like(h_scr)

        x = emb_ref[0]
        h = h_scr[...]
        gi = jnp.dot(x, wih_ref[0], preferred_element_type=jnp.float32) + bih_ref[0, 0]
        gh = jnp.dot(h, whh_ref[0], preferred_element_type=jnp.float32) + bhh_ref[0, 0]
        i_r, i_z, i_n = gi[:, :H], gi[:, H:2 * H], gi[:, 2 * H:]
        h_r, h_z, h_n = gh[:, :H], gh[:, H:2 * H], gh[:, 2 * H:]
        r = jax.nn.sigmoid(i_r + h_r)
        z = jax.nn.sigmoid(i_z + h_z)
        n = jnp.tanh(i_n + r * h_n)
        h_new = (1.0 - z) * n + z * h
        h_scr[...] = h_new
        out_ref[0, 0] = h_new
        hid_ref[0] = h_new

    t_of = lambda i: jnp.where(i < T, i, 2 * T - 1 - i)
    d_of = lambda i: jnp.where(i < T, 0, 1)

    out, hid = pl.pallas_call(
        body,
        grid=(2 * T,),
        in_specs=[
            pl.BlockSpec((1, B, HP), lambda i: (t_of(i), 0, 0)),
            pl.BlockSpec((1, HP, H3), lambda i: (d_of(i), 0, 0)),
            pl.BlockSpec((1, H, H3), lambda i: (d_of(i), 0, 0)),
            pl.BlockSpec((1, 1, H3), lambda i: (d_of(i), 0, 0)),
            pl.BlockSpec((1, 1, H3), lambda i: (d_of(i), 0, 0)),
        ],
        out_specs=[
            pl.BlockSpec((1, 1, B, H), lambda i: (d_of(i), t_of(i), 0, 0)),
            pl.BlockSpec((1, B, H), lambda i: (d_of(i), 0, 0)),
        ],
        out_shape=[
            jax.ShapeDtypeStruct((2, T, B, H), jnp.float32),
            jax.ShapeDtypeStruct((2, B, H), jnp.float32),
        ],
        scratch_shapes=[pltpu.VMEM((B, H), jnp.float32)],
        compiler_params=pltpu.CompilerParams(
            dimension_semantics=("arbitrary",)
        ),
    )(emb, w_ih_t, w_hh_t, b_ih, b_hh)
    return out, hid


def kernel(input_seqs, input_lengths, table, w_ih_f, w_hh_f, b_ih_f, b_hh_f,
           w_ih_b, w_hh_b, b_ih_b, b_hh_b):
    T, B = input_seqs.shape
    V, H = table.shape
    HP = ((H + 127) // 128) * 128
    table_p = jnp.pad(table, ((0, 0), (0, HP - H)))
    emb = _sc_gather(table_p, input_seqs.reshape(T * B)).reshape(T, B, HP)
    w_ih_t = jnp.pad(jnp.stack([w_ih_f.T, w_ih_b.T]), ((0, 0), (0, HP - H), (0, 0)))
    w_hh_t = jnp.stack([w_hh_f.T, w_hh_b.T])
    b_ih = jnp.stack([b_ih_f, b_ih_b])[:, None, :]
    b_hh = jnp.stack([b_hh_f, b_hh_b])[:, None, :]
    out2, hid = _gru_bidir_tc(emb, w_ih_t, w_hh_t, b_ih, b_hh)
    return jnp.concatenate([out2[0], out2[1]], axis=-1), hid


# speedup vs baseline: 1.9179x; 1.4280x over previous
import functools

import jax
import jax.numpy as jnp
from jax import lax
from jax.experimental import pallas as pl
from jax.experimental.pallas import tpu as pltpu
from jax.experimental.pallas import tpu_sc as plsc


def _tc_tail(table, C0, TW):
    V, H = table.shape
    W = H - C0
    R = 2000
    assert V % R == 0 and C0 % 128 == 0 and W <= TW

    def body(tab_ref, out_ref):
        lane = lax.broadcasted_iota(jnp.int32, (R, TW), 1)
        out_ref[...] = jnp.where(lane < W, tab_ref[...], 0.0)

    return pl.pallas_call(
        body,
        grid=(V // R,),
        in_specs=[pl.BlockSpec((R, TW), lambda i: (i, C0 // TW))],
        out_specs=pl.BlockSpec((R, TW), lambda i: (i, 0)),
        out_shape=jax.ShapeDtypeStruct((V, TW), jnp.float32),
        compiler_params=pltpu.CompilerParams(
            dimension_semantics=("arbitrary",)
        ),
    )(table)


def _sc_gather(table, tail, idx):
    V, H = table.shape
    TW = tail.shape[1]
    C0 = 256
    HP = C0 + TW
    N = idx.shape[0]
    info = plsc.get_sparse_core_info()
    NC, NS = info.num_cores, info.num_subcores
    NW = NC * NS
    assert N % NW == 0
    per_w = N // NW
    CH = 128
    sizes = [CH] * (per_w // CH)
    if per_w % CH:
        sizes.append(per_w % CH)
    offs = [0]
    for s in sizes[:-1]:
        offs.append(offs[-1] + s)
    n_ch = len(sizes)

    mesh = plsc.VectorSubcoreMesh(core_axis_name="c", subcore_axis_name="s")

    @functools.partial(
        pl.kernel,
        mesh=mesh,
        out_type=jax.ShapeDtypeStruct((N, HP), jnp.float32),
        scratch_types=[
            pltpu.VMEM((per_w,), jnp.int32),
            pltpu.VMEM((CH, HP), jnp.float32),
            pltpu.VMEM((CH, HP), jnp.float32),
            pltpu.SemaphoreType.DMA,
            pltpu.SemaphoreType.DMA,
        ],
    )
    def gather_kernel(table_hbm, tail_hbm, idx_hbm, out_hbm, idx_v, rows0, rows1,
                      sem0, sem1):
        wid = lax.axis_index("s") * NC + lax.axis_index("c")
        base = wid * per_w
        pltpu.sync_copy(idx_hbm.at[pl.ds(base, per_w)], idx_v)
        bufs = (rows0, rows1)
        sems = (sem0, sem1)

        def start(j):
            n = sizes[j]
            ids = idx_v.at[pl.ds(offs[j], n)]
            buf = bufs[j % 2]
            sem = sems[j % 2]
            cp_a = pltpu.async_copy(
                table_hbm.at[ids, pl.ds(0, C0)],
                buf.at[pl.ds(0, n), pl.ds(0, C0)], sem)
            cp_b = pltpu.async_copy(
                tail_hbm.at[ids],
                buf.at[pl.ds(0, n), pl.ds(C0, TW)], sem)
            return (cp_a, cp_b)

        def drain(j, cps):
            cps[0].wait()
            cps[1].wait()
            pltpu.sync_copy(
                bufs[j % 2].at[pl.ds(0, sizes[j])],
                out_hbm.at[pl.ds(base + offs[j], sizes[j])],
            )

        cp = start(0)
        for j in range(1, n_ch):
            cp_next = start(j)
            drain(j - 1, cp)
            cp = cp_next
        drain(n_ch - 1, cp)

    return gather_kernel(table, tail, idx)


def _gru_bidir_tc(emb, w_ih_t, w_hh_t, b_ih, b_hh):
    T, B, HP = emb.shape
    H = w_hh_t.shape[1]
    H3 = 3 * H

    def body(emb_ref, wih_ref, whh_ref, bih_ref, bhh_ref, out_ref, hid_ref, h_scr):
        i = pl.program_id(0)

        @pl.when((i == 0) | (i == T))
        def _():
            h_scr[...] = jnp.zeros_like(h_scr)

        x = emb_ref[0]
        h = h_scr[...]
        gi = jnp.dot(x, wih_ref[0], preferred_element_type=jnp.float32) + bih_ref[0, 0]
        gh = jnp.dot(h, whh_ref[0], preferred_element_type=jnp.float32) + bhh_ref[0, 0]
        i_r, i_z, i_n = gi[:, :H], gi[:, H:2 * H], gi[:, 2 * H:]
        h_r, h_z, h_n = gh[:, :H], gh[:, H:2 * H], gh[:, 2 * H:]
        r = jax.nn.sigmoid(i_r + h_r)
        z = jax.nn.sigmoid(i_z + h_z)
        n = jnp.tanh(i_n + r * h_n)
        h_new = (1.0 - z) * n + z * h
        h_scr[...] = h_new
        out_ref[0, 0] = h_new
        hid_ref[0] = h_new

    t_of = lambda i: jnp.where(i < T, i, 2 * T - 1 - i)
    d_of = lambda i: jnp.where(i < T, 0, 1)

    out, hid = pl.pallas_call(
        body,
        grid=(2 * T,),
        in_specs=[
            pl.BlockSpec((1, B, HP), lambda i: (t_of(i), 0, 0)),
            pl.BlockSpec((1, HP, H3), lambda i: (d_of(i), 0, 0)),
            pl.BlockSpec((1, H, H3), lambda i: (d_of(i), 0, 0)),
            pl.BlockSpec((1, 1, H3), lambda i: (d_of(i), 0, 0)),
            pl.BlockSpec((1, 1, H3), lambda i: (d_of(i), 0, 0)),
        ],
        out_specs=[
            pl.BlockSpec((1, 1, B, H), lambda i: (d_of(i), t_of(i), 0, 0)),
            pl.BlockSpec((1, B, H), lambda i: (d_of(i), 0, 0)),
        ],
        out_shape=[
            jax.ShapeDtypeStruct((2, T, B, H), jnp.float32),
            jax.ShapeDtypeStruct((2, B, H), jnp.float32),
        ],
        scratch_shapes=[pltpu.VMEM((B, H), jnp.float32)],
        compiler_params=pltpu.CompilerParams(
            dimension_semantics=("arbitrary",)
        ),
    )(emb, w_ih_t, w_hh_t, b_ih, b_hh)
    return out, hid


def kernel(input_seqs, input_lengths, table, w_ih_f, w_hh_f, b_ih_f, b_hh_f,
           w_ih_b, w_hh_b, b_ih_b, b_hh_b):
    T, B = input_seqs.shape
    V, H = table.shape
    C0, TW = 256, 128
    HP = C0 + TW
    tail = _tc_tail(table, C0, TW)
    emb = _sc_gather(table, tail, input_seqs.reshape(T * B)).reshape(T, B, HP)
    w_ih_t = jnp.pad(jnp.stack([w_ih_f.T, w_ih_b.T]), ((0, 0), (0, HP - H), (0, 0)))
    w_hh_t = jnp.stack([w_hh_f.T, w_hh_b.T])
    b_ih = jnp.stack([b_ih_f, b_ih_b])[:, None, :]
    b_hh = jnp.stack([b_hh_f, b_hh_b])[:, None, :]
    out2, hid = _gru_bidir_tc(emb, w_ih_t, w_hh_t, b_ih, b_hh)
    return jnp.concatenate([out2[0], out2[1]], axis=-1), hid
